# baseline (speedup 1.0000x reference)
import functools

import jax
import jax.numpy as jnp
from jax import lax
from jax.experimental import pallas as pl
from jax.experimental.pallas import tpu as pltpu
from jax.experimental.pallas import tpu_sc as plsc

B = 4096
N = 20000

RB = 512
CB = 2048
NR = B // RB
NC = (N + CB - 1) // CB

NEG_INF = float("-inf")
BIG_I32 = 2**31 - 1

SC_CORES = 2
SC_SUBCORES = 16
SC_LANES = 16
NW = SC_CORES * SC_SUBCORES
RPW = B // NW


def _reduce_zero_body(x_ref, alloc_ref, pay_ref, fi_ref, fp_ref,
                      m_s, s_s, i_s):
    r = pl.program_id(0)
    c = pl.program_id(1)

    @pl.when(r >= 1)
    def _fill():
        alloc_ref[...] = jnp.zeros_like(alloc_ref)
        pay_ref[...] = jnp.zeros_like(pay_ref)

    @pl.when(r < NR)
    def _reduce():
        @pl.when(c == 0)
        def _init():
            m_s[...] = jnp.full(m_s.shape, NEG_INF, m_s.dtype)
            s_s[...] = jnp.full(s_s.shape, NEG_INF, s_s.dtype)
            i_s[...] = jnp.zeros(i_s.shape, i_s.dtype)

        gcol = c * CB + lax.broadcasted_iota(jnp.int32, (RB, CB), 1)
        x = jnp.where(gcol < N, x_ref[...], NEG_INF)
        m_blk = jnp.max(x, axis=1, keepdims=True)
        idx_blk = jnp.min(jnp.where(x == m_blk, gcol, BIG_I32), axis=1,
                          keepdims=True)
        s_blk = jnp.max(jnp.where(gcol == idx_blk, NEG_INF, x), axis=1,
                        keepdims=True)

        m_run, s_run, i_run = m_s[...], s_s[...], i_s[...]
        m_s[...] = jnp.maximum(m_run, m_blk)
        s_s[...] = jnp.maximum(jnp.maximum(s_run, s_blk),
                               jnp.minimum(m_run, m_blk))
        i_s[...] = jnp.where(m_blk > m_run, idx_blk, i_run)

        @pl.when(c == NC - 1)
        def _finalize():
            fi_ref[...] = jnp.broadcast_to(i_s[...], fi_ref.shape)
            fp_ref[...] = jnp.broadcast_to(jnp.maximum(s_s[...], 0.0),
                                           fp_ref.shape)


def _tc_reduce_and_zero(virtual_values):
    return pl.pallas_call(
        _reduce_zero_body,
        grid=(NR + 1, NC),
        in_specs=[
            pl.BlockSpec(
                (RB, CB),
                lambda r, c: (jnp.minimum(r, NR - 1),
                              jnp.where(r < NR, c, NC - 1)),
            ),
        ],
        out_specs=[
            pl.BlockSpec(
                (RB, CB),
                lambda r, c: (jnp.maximum(r - 1, 0),
                              jnp.where(r >= 1, c, 0)),
            ),
            pl.BlockSpec(
                (RB, CB),
                lambda r, c: (jnp.maximum(r - 1, 0),
                              jnp.where(r >= 1, c, 0)),
            ),
            pl.BlockSpec((RB, 128),
                         lambda r, c: (jnp.minimum(r, NR - 1), 0)),
            pl.BlockSpec((RB, 128),
                         lambda r, c: (jnp.minimum(r, NR - 1), 0)),
        ],
        out_shape=[
            jax.ShapeDtypeStruct((B, N), jnp.float32),
            jax.ShapeDtypeStruct((B, N), jnp.float32),
            jax.ShapeDtypeStruct((B, 128), jnp.int32),
            jax.ShapeDtypeStruct((B, 128), jnp.float32),
        ],
        scratch_shapes=[
            pltpu.VMEM((RB, 1), jnp.float32),
            pltpu.VMEM((RB, 1), jnp.float32),
            pltpu.VMEM((RB, 1), jnp.int32),
        ],
        compiler_params=pltpu.CompilerParams(
            dimension_semantics=("arbitrary", "arbitrary"),
        ),
    )(virtual_values)


def _sc_scatter(alloc_flat, pay_flat, fi, fp):
    mesh = plsc.VectorSubcoreMesh(core_axis_name="c", subcore_axis_name="s")

    def stateful(refs):
        alloc_ref, pay_ref, fi_ref, fp_ref = refs

        @pl.core_map(mesh)
        def _():
            wid = lax.axis_index("s") * SC_CORES + lax.axis_index("c")
            base = wid * RPW

            def inner(fib, fpb, addrb, onesb, sem):
                pltpu.async_copy(fi_ref.at[wid], fib, sem).wait()
                pltpu.async_copy(fp_ref.at[wid], fpb, sem).wait()
                lane = lax.iota(jnp.int32, SC_LANES)
                for k in range(RPW // SC_LANES):
                    rows16 = base + k * SC_LANES + lane
                    win = fib[pl.ds(k * SC_LANES, SC_LANES)]
                    addrb[pl.ds(k * SC_LANES, SC_LANES)] = rows16 * N + win
                    onesb[pl.ds(k * SC_LANES, SC_LANES)] = jnp.full(
                        (SC_LANES,), 1.0, jnp.float32)
                pltpu.async_copy(onesb, alloc_ref.at[addrb], sem).wait()
                pltpu.async_copy(fpb, pay_ref.at[addrb], sem).wait()

            pl.run_scoped(
                inner,
                pltpu.VMEM((RPW,), jnp.int32),
                pltpu.VMEM((RPW,), jnp.float32),
                pltpu.VMEM((RPW,), jnp.int32),
                pltpu.VMEM((RPW,), jnp.float32),
                pltpu.SemaphoreType.DMA,
            )

    return pl.run_state(stateful)((alloc_flat, pay_flat, fi, fp))


@jax.jit
def kernel(virtual_values):
    alloc_z, pay_z, fi, fp = _tc_reduce_and_zero(virtual_values)
    fi_t = fi[:, 0].reshape(NW, RPW)
    fp_t = fp[:, 0].reshape(NW, RPW)
    alloc_flat, pay_flat, _, _ = _sc_scatter(
        alloc_z.reshape(B * N), pay_z.reshape(B * N), fi_t, fp_t)
    return (alloc_flat.reshape(B, N), pay_flat.reshape(B, N))


# speedup vs baseline: 1.7913x; 1.7913x over previous
import functools

import jax
import jax.numpy as jnp
from jax import lax
from jax.experimental import pallas as pl
from jax.experimental.pallas import tpu as pltpu
from jax.experimental.pallas import tpu_sc as plsc

B = 4096
N = 20000

RB = 512
CB = 2048
NR = B // RB
NC = (N + CB - 1) // CB

NEG_INF = float("-inf")
BIG_I32 = 2**31 - 1

SC_CORES = 2
SC_SUBCORES = 16
SC_LANES = 16
NW = SC_CORES * SC_SUBCORES
RPW = B // NW


def _reduce_zero_body(x_ref, alloc_ref, pay_ref, fi_ref, fp_ref,
                      m_s, s_s, i_s):
    r = pl.program_id(0)
    c = pl.program_id(1)

    @pl.when(r >= 1)
    def _fill():
        alloc_ref[...] = jnp.zeros_like(alloc_ref)
        pay_ref[...] = jnp.zeros_like(pay_ref)

    @pl.when(r < NR)
    def _reduce():
        @pl.when(c == 0)
        def _init():
            m_s[...] = jnp.full(m_s.shape, NEG_INF, m_s.dtype)
            s_s[...] = jnp.full(s_s.shape, NEG_INF, s_s.dtype)
            i_s[...] = jnp.zeros(i_s.shape, i_s.dtype)

        gcol = c * CB + lax.broadcasted_iota(jnp.int32, (RB, CB), 1)
        x = jnp.where(gcol < N, x_ref[...], NEG_INF)
        m_blk = jnp.max(x, axis=1, keepdims=True)
        idx_blk = jnp.min(jnp.where(x == m_blk, gcol, BIG_I32), axis=1,
                          keepdims=True)
        s_blk = jnp.max(jnp.where(gcol == idx_blk, NEG_INF, x), axis=1,
                        keepdims=True)

        m_run, s_run, i_run = m_s[...], s_s[...], i_s[...]
        m_s[...] = jnp.maximum(m_run, m_blk)
        s_s[...] = jnp.maximum(jnp.maximum(s_run, s_blk),
                               jnp.minimum(m_run, m_blk))
        i_s[...] = jnp.where(m_blk > m_run, idx_blk, i_run)

        @pl.when(c == NC - 1)
        def _finalize():
            fi_ref[...] = jnp.broadcast_to(i_s[...], fi_ref.shape)
            fp_ref[...] = jnp.broadcast_to(jnp.maximum(s_s[...], 0.0),
                                           fp_ref.shape)


def _tc_reduce_and_zero(virtual_values):
    return pl.pallas_call(
        _reduce_zero_body,
        grid=(NR + 1, NC),
        in_specs=[
            pl.BlockSpec(
                (RB, CB),
                lambda r, c: (jnp.minimum(r, NR - 1),
                              jnp.where(r < NR, c, NC - 1)),
            ),
        ],
        out_specs=[
            pl.BlockSpec(
                (RB, CB),
                lambda r, c: (jnp.maximum(r - 1, 0),
                              jnp.where(r >= 1, c, 0)),
            ),
            pl.BlockSpec(
                (RB, CB),
                lambda r, c: (jnp.maximum(r - 1, 0),
                              jnp.where(r >= 1, c, 0)),
            ),
            pl.BlockSpec((RB, 128),
                         lambda r, c: (jnp.minimum(r, NR - 1), 0)),
            pl.BlockSpec((RB, 128),
                         lambda r, c: (jnp.minimum(r, NR - 1), 0)),
        ],
        out_shape=[
            jax.ShapeDtypeStruct((B, N), jnp.float32),
            jax.ShapeDtypeStruct((B, N), jnp.float32),
            jax.ShapeDtypeStruct((B, 128), jnp.int32),
            jax.ShapeDtypeStruct((B, 128), jnp.float32),
        ],
        scratch_shapes=[
            pltpu.VMEM((RB, 1), jnp.float32),
            pltpu.VMEM((RB, 1), jnp.float32),
            pltpu.VMEM((RB, 1), jnp.int32),
        ],
        compiler_params=pltpu.CompilerParams(
            dimension_semantics=("arbitrary", "arbitrary"),
        ),
    )(virtual_values)


def _sc_scatter(alloc_flat, pay_flat, fi, fp):
    mesh = plsc.VectorSubcoreMesh(core_axis_name="c", subcore_axis_name="s")

    def stateful(refs):
        alloc_ref, pay_ref, fi_ref, fp_ref = refs

        @pl.core_map(mesh)
        def _():
            wid = lax.axis_index("s") * SC_CORES + lax.axis_index("c")
            base = wid * RPW

            def inner(fib, fpb, stage_a, stage_p, sem):
                pltpu.async_copy(fi_ref.at[wid], fib, sem).wait()
                pltpu.async_copy(fp_ref.at[wid], fpb, sem).wait()
                lane = lax.iota(jnp.int32, SC_LANES)
                handles = []
                for k in range(RPW // SC_LANES):
                    wv = fib[pl.ds(k * SC_LANES, SC_LANES)]
                    pv = fpb[pl.ds(k * SC_LANES, SC_LANES)]
                    for j in range(SC_LANES):
                        jl = k * SC_LANES + j
                        col = wv[j]
                        w16 = (col // SC_LANES) * SC_LANES
                        off = col - w16
                        sl = pl.ds(jl * SC_LANES, SC_LANES)
                        stage_a[sl] = jnp.where(lane == off, 1.0,
                                                0.0).astype(jnp.float32)
                        stage_p[sl] = jnp.where(lane == off, pv[j],
                                                0.0).astype(jnp.float32)
                        row = base + jl
                        dst = pl.ds(w16, SC_LANES)
                        handles.append(pltpu.async_copy(
                            stage_a.at[sl], alloc_ref.at[row, dst], sem))
                        handles.append(pltpu.async_copy(
                            stage_p.at[sl], pay_ref.at[row, dst], sem))
                for h in handles:
                    h.wait()

            pl.run_scoped(
                inner,
                pltpu.VMEM((RPW,), jnp.int32),
                pltpu.VMEM((RPW,), jnp.float32),
                pltpu.VMEM((RPW * SC_LANES,), jnp.float32),
                pltpu.VMEM((RPW * SC_LANES,), jnp.float32),
                pltpu.SemaphoreType.DMA,
            )

    return pl.run_state(stateful)((alloc_flat, pay_flat, fi, fp))


@jax.jit
def kernel(virtual_values):
    alloc_z, pay_z, fi, fp = _tc_reduce_and_zero(virtual_values)
    fi_t = fi[:, 0].reshape(NW, RPW)
    fp_t = fp[:, 0].reshape(NW, RPW)
    allocations, payments, _, _ = _sc_scatter(alloc_z, pay_z, fi_t, fp_t)
    return (allocations, payments)


# speedup vs baseline: 1.7962x; 1.0027x over previous
import functools

import jax
import jax.numpy as jnp
from jax import lax
from jax.experimental import pallas as pl
from jax.experimental.pallas import tpu as pltpu
from jax.experimental.pallas import tpu_sc as plsc

B = 4096
N = 20000

RB = 512
CB = 2560
NR = B // RB
NC = (N + CB - 1) // CB

NEG_INF = float("-inf")
BIG_I32 = 2**31 - 1

SC_CORES = 2
SC_SUBCORES = 16
SC_LANES = 16
NW = SC_CORES * SC_SUBCORES
RPW = B // NW


def _reduce_zero_body(x_ref, alloc_ref, pay_ref, fi_ref, fp_ref,
                      m_s, s_s, i_s):
    r = pl.program_id(0)
    c = pl.program_id(1)

    @pl.when(r >= 1)
    def _fill():
        alloc_ref[...] = jnp.zeros_like(alloc_ref)
        pay_ref[...] = jnp.zeros_like(pay_ref)

    @pl.when(r < NR)
    def _reduce():
        @pl.when(c == 0)
        def _init():
            m_s[...] = jnp.full(m_s.shape, NEG_INF, m_s.dtype)
            s_s[...] = jnp.full(s_s.shape, NEG_INF, s_s.dtype)
            i_s[...] = jnp.zeros(i_s.shape, i_s.dtype)

        gcol = c * CB + lax.broadcasted_iota(jnp.int32, (RB, CB), 1)
        x = jnp.where(gcol < N, x_ref[...], NEG_INF)
        m_blk = jnp.max(x, axis=1, keepdims=True)
        idx_blk = jnp.min(jnp.where(x == m_blk, gcol, BIG_I32), axis=1,
                          keepdims=True)
        s_blk = jnp.max(jnp.where(gcol == idx_blk, NEG_INF, x), axis=1,
                        keepdims=True)

        m_run, s_run, i_run = m_s[...], s_s[...], i_s[...]
        m_s[...] = jnp.maximum(m_run, m_blk)
        s_s[...] = jnp.maximum(jnp.maximum(s_run, s_blk),
                               jnp.minimum(m_run, m_blk))
        i_s[...] = jnp.where(m_blk > m_run, idx_blk, i_run)

        @pl.when(c == NC - 1)
        def _finalize():
            fi_ref[...] = jnp.broadcast_to(i_s[...], fi_ref.shape)
            fp_ref[...] = jnp.broadcast_to(jnp.maximum(s_s[...], 0.0),
                                           fp_ref.shape)


def _tc_reduce_and_zero(virtual_values):
    return pl.pallas_call(
        _reduce_zero_body,
        grid=(NR + 1, NC),
        in_specs=[
            pl.BlockSpec(
                (RB, CB),
                lambda r, c: (jnp.minimum(r, NR - 1),
                              jnp.where(r < NR, c, NC - 1)),
            ),
        ],
        out_specs=[
            pl.BlockSpec(
                (RB, CB),
                lambda r, c: (jnp.maximum(r - 1, 0),
                              jnp.where(r >= 1, c, 0)),
            ),
            pl.BlockSpec(
                (RB, CB),
                lambda r, c: (jnp.maximum(r - 1, 0),
                              jnp.where(r >= 1, c, 0)),
            ),
            pl.BlockSpec((RB, 128),
                         lambda r, c: (jnp.minimum(r, NR - 1), 0)),
            pl.BlockSpec((RB, 128),
                         lambda r, c: (jnp.minimum(r, NR - 1), 0)),
        ],
        out_shape=[
            jax.ShapeDtypeStruct((B, N), jnp.float32),
            jax.ShapeDtypeStruct((B, N), jnp.float32),
            jax.ShapeDtypeStruct((B, 128), jnp.int32),
            jax.ShapeDtypeStruct((B, 128), jnp.float32),
        ],
        scratch_shapes=[
            pltpu.VMEM((RB, 1), jnp.float32),
            pltpu.VMEM((RB, 1), jnp.float32),
            pltpu.VMEM((RB, 1), jnp.int32),
        ],
        compiler_params=pltpu.CompilerParams(
            dimension_semantics=("arbitrary", "arbitrary"),
        ),
    )(virtual_values)


def _sc_scatter(alloc_flat, pay_flat, fi, fp):
    mesh = plsc.VectorSubcoreMesh(core_axis_name="c", subcore_axis_name="s")

    def stateful(refs):
        alloc_ref, pay_ref, fi_ref, fp_ref = refs

        @pl.core_map(mesh)
        def _():
            wid = lax.axis_index("s") * SC_CORES + lax.axis_index("c")
            base = wid * RPW

            def inner(fib, fpb, stage_a, stage_p, sem):
                pltpu.async_copy(fi_ref.at[wid], fib, sem).wait()
                pltpu.async_copy(fp_ref.at[wid], fpb, sem).wait()
                lane = lax.iota(jnp.int32, SC_LANES)
                handles = []
                for k in range(RPW // SC_LANES):
                    wv = fib[pl.ds(k * SC_LANES, SC_LANES)]
                    pv = fpb[pl.ds(k * SC_LANES, SC_LANES)]
                    for j in range(SC_LANES):
                        jl = k * SC_LANES + j
                        col = wv[j]
                        w16 = (col // SC_LANES) * SC_LANES
                        off = col - w16
                        sl = pl.ds(jl * SC_LANES, SC_LANES)
                        stage_a[sl] = jnp.where(lane == off, 1.0,
                                                0.0).astype(jnp.float32)
                        stage_p[sl] = jnp.where(lane == off, pv[j],
                                                0.0).astype(jnp.float32)
                        row = base + jl
                        dst = pl.ds(w16, SC_LANES)
                        handles.append(pltpu.async_copy(
                            stage_a.at[sl], alloc_ref.at[row, dst], sem))
                        handles.append(pltpu.async_copy(
                            stage_p.at[sl], pay_ref.at[row, dst], sem))
                for h in handles:
                    h.wait()

            pl.run_scoped(
                inner,
                pltpu.VMEM((RPW,), jnp.int32),
                pltpu.VMEM((RPW,), jnp.float32),
                pltpu.VMEM((RPW * SC_LANES,), jnp.float32),
                pltpu.VMEM((RPW * SC_LANES,), jnp.float32),
                pltpu.SemaphoreType.DMA,
            )

    return pl.run_state(stateful)((alloc_flat, pay_flat, fi, fp))


@jax.jit
def kernel(virtual_values):
    alloc_z, pay_z, fi, fp = _tc_reduce_and_zero(virtual_values)
    fi_t = fi[:, 0].reshape(NW, RPW)
    fp_t = fp[:, 0].reshape(NW, RPW)
    allocations, payments, _, _ = _sc_scatter(alloc_z, pay_z, fi_t, fp_t)
    return (allocations, payments)


# speedup vs baseline: 1.7990x; 1.0016x over previous
import functools

import jax
import jax.numpy as jnp
from jax import lax
from jax.experimental import pallas as pl
from jax.experimental.pallas import tpu as pltpu
from jax.experimental.pallas import tpu_sc as plsc

B = 4096
N = 20000

RB = 256
CB = 5120
NR = B // RB
NC = (N + CB - 1) // CB

NEG_INF = float("-inf")
BIG_I32 = 2**31 - 1

SC_CORES = 2
SC_SUBCORES = 16
SC_LANES = 16
NW = SC_CORES * SC_SUBCORES
RPW = B // NW


def _reduce_zero_body(x_ref, alloc_ref, pay_ref, fi_ref, fp_ref,
                      m_s, s_s, i_s):
    r = pl.program_id(0)
    c = pl.program_id(1)

    @pl.when(r >= 1)
    def _fill():
        alloc_ref[...] = jnp.zeros_like(alloc_ref)
        pay_ref[...] = jnp.zeros_like(pay_ref)

    @pl.when(r < NR)
    def _reduce():
        @pl.when(c == 0)
        def _init():
            m_s[...] = jnp.full(m_s.shape, NEG_INF, m_s.dtype)
            s_s[...] = jnp.full(s_s.shape, NEG_INF, s_s.dtype)
            i_s[...] = jnp.zeros(i_s.shape, i_s.dtype)

        gcol = c * CB + lax.broadcasted_iota(jnp.int32, (RB, CB), 1)
        x = jnp.where(gcol < N, x_ref[...], NEG_INF)
        m_blk = jnp.max(x, axis=1, keepdims=True)
        idx_blk = jnp.min(jnp.where(x == m_blk, gcol, BIG_I32), axis=1,
                          keepdims=True)
        s_blk = jnp.max(jnp.where(gcol == idx_blk, NEG_INF, x), axis=1,
                        keepdims=True)

        m_run, s_run, i_run = m_s[...], s_s[...], i_s[...]
        m_s[...] = jnp.maximum(m_run, m_blk)
        s_s[...] = jnp.maximum(jnp.maximum(s_run, s_blk),
                               jnp.minimum(m_run, m_blk))
        i_s[...] = jnp.where(m_blk > m_run, idx_blk, i_run)

        @pl.when(c == NC - 1)
        def _finalize():
            fi_ref[...] = jnp.broadcast_to(i_s[...], fi_ref.shape)
            fp_ref[...] = jnp.broadcast_to(jnp.maximum(s_s[...], 0.0),
                                           fp_ref.shape)


def _tc_reduce_and_zero(virtual_values):
    return pl.pallas_call(
        _reduce_zero_body,
        grid=(NR + 1, NC),
        in_specs=[
            pl.BlockSpec(
                (RB, CB),
                lambda r, c: (jnp.minimum(r, NR - 1),
                              jnp.where(r < NR, c, NC - 1)),
            ),
        ],
        out_specs=[
            pl.BlockSpec(
                (RB, CB),
                lambda r, c: (jnp.maximum(r - 1, 0),
                              jnp.where(r >= 1, c, 0)),
            ),
            pl.BlockSpec(
                (RB, CB),
                lambda r, c: (jnp.maximum(r - 1, 0),
                              jnp.where(r >= 1, c, 0)),
            ),
            pl.BlockSpec((RB, 128),
                         lambda r, c: (jnp.minimum(r, NR - 1), 0)),
            pl.BlockSpec((RB, 128),
                         lambda r, c: (jnp.minimum(r, NR - 1), 0)),
        ],
        out_shape=[
            jax.ShapeDtypeStruct((B, N), jnp.float32),
            jax.ShapeDtypeStruct((B, N), jnp.float32),
            jax.ShapeDtypeStruct((B, 128), jnp.int32),
            jax.ShapeDtypeStruct((B, 128), jnp.float32),
        ],
        scratch_shapes=[
            pltpu.VMEM((RB, 1), jnp.float32),
            pltpu.VMEM((RB, 1), jnp.float32),
            pltpu.VMEM((RB, 1), jnp.int32),
        ],
        compiler_params=pltpu.CompilerParams(
            dimension_semantics=("arbitrary", "arbitrary"),
        ),
    )(virtual_values)


def _sc_scatter(alloc_flat, pay_flat, fi, fp):
    mesh = plsc.VectorSubcoreMesh(core_axis_name="c", subcore_axis_name="s")

    def stateful(refs):
        alloc_ref, pay_ref, fi_ref, fp_ref = refs

        @pl.core_map(mesh)
        def _():
            wid = lax.axis_index("s") * SC_CORES + lax.axis_index("c")
            base = wid * RPW

            def inner(fib, fpb, stage_a, stage_p, sem):
                pltpu.async_copy(fi_ref.at[wid], fib, sem).wait()
                pltpu.async_copy(fp_ref.at[wid], fpb, sem).wait()
                lane = lax.iota(jnp.int32, SC_LANES)
                handles = []
                for k in range(RPW // SC_LANES):
                    wv = fib[pl.ds(k * SC_LANES, SC_LANES)]
                    pv = fpb[pl.ds(k * SC_LANES, SC_LANES)]
                    for j in range(SC_LANES):
                        jl = k * SC_LANES + j
                        col = wv[j]
                        w16 = (col // SC_LANES) * SC_LANES
                        off = col - w16
                        sl = pl.ds(jl * SC_LANES, SC_LANES)
                        stage_a[sl] = jnp.where(lane == off, 1.0,
                                                0.0).astype(jnp.float32)
                        stage_p[sl] = jnp.where(lane == off, pv[j],
                                                0.0).astype(jnp.float32)
                        row = base + jl
                        dst = pl.ds(w16, SC_LANES)
                        handles.append(pltpu.async_copy(
                            stage_a.at[sl], alloc_ref.at[row, dst], sem))
                        handles.append(pltpu.async_copy(
                            stage_p.at[sl], pay_ref.at[row, dst], sem))
                for h in handles:
                    h.wait()

            pl.run_scoped(
                inner,
                pltpu.VMEM((RPW,), jnp.int32),
                pltpu.VMEM((RPW,), jnp.float32),
                pltpu.VMEM((RPW * SC_LANES,), jnp.float32),
                pltpu.VMEM((RPW * SC_LANES,), jnp.float32),
                pltpu.SemaphoreType.DMA,
            )

    return pl.run_state(stateful)((alloc_flat, pay_flat, fi, fp))


@jax.jit
def kernel(virtual_values):
    alloc_z, pay_z, fi, fp = _tc_reduce_and_zero(virtual_values)
    fi_t = fi[:, 0].reshape(NW, RPW)
    fp_t = fp[:, 0].reshape(NW, RPW)
    allocations, payments, _, _ = _sc_scatter(alloc_z, pay_z, fi_t, fp_t)
    return (allocations, payments)


# speedup vs baseline: 1.7995x; 1.0003x over previous
import functools

import jax
import jax.numpy as jnp
from jax import lax
from jax.experimental import pallas as pl
from jax.experimental.pallas import tpu as pltpu
from jax.experimental.pallas import tpu_sc as plsc

B = 4096
N = 20000

RB = 256
CB = 5120
NR = B // RB
NC = (N + CB - 1) // CB

NEG_INF = float("-inf")
BIG_I32 = 2**31 - 1

SC_CORES = 2
SC_SUBCORES = 16
SC_LANES = 16
NW = SC_CORES * SC_SUBCORES
RPW = B // NW


def _reduce_zero_body(x_ref, alloc_ref, pay_ref, fi_ref, fp_ref,
                      m_s, s_s, i_s):
    r = pl.program_id(0)
    c = pl.program_id(1)

    @pl.when(r >= 1)
    def _fill():
        alloc_ref[...] = jnp.zeros_like(alloc_ref)
        pay_ref[...] = jnp.zeros_like(pay_ref)

    @pl.when(r < NR)
    def _reduce():
        @pl.when(c == 0)
        def _init():
            m_s[...] = jnp.full(m_s.shape, NEG_INF, m_s.dtype)
            s_s[...] = jnp.full(s_s.shape, NEG_INF, s_s.dtype)
            i_s[...] = jnp.zeros(i_s.shape, i_s.dtype)

        gcol = c * CB + lax.broadcasted_iota(jnp.int32, (RB, CB), 1)
        x = jnp.where(gcol < N, x_ref[...], NEG_INF)
        m_blk = jnp.max(x, axis=1, keepdims=True)
        idx_blk = jnp.min(jnp.where(x == m_blk, gcol, BIG_I32), axis=1,
                          keepdims=True)
        s_blk = jnp.max(jnp.where(gcol == idx_blk, NEG_INF, x), axis=1,
                        keepdims=True)

        m_run, s_run, i_run = m_s[...], s_s[...], i_s[...]
        m_s[...] = jnp.maximum(m_run, m_blk)
        s_s[...] = jnp.maximum(jnp.maximum(s_run, s_blk),
                               jnp.minimum(m_run, m_blk))
        i_s[...] = jnp.where(m_blk > m_run, idx_blk, i_run)

        @pl.when(c == NC - 1)
        def _finalize():
            fi_ref[...] = jnp.broadcast_to(i_s[...], fi_ref.shape)
            fp_ref[...] = jnp.broadcast_to(jnp.maximum(s_s[...], 0.0),
                                           fp_ref.shape)


def _tc_reduce_and_zero(virtual_values):
    return pl.pallas_call(
        _reduce_zero_body,
        grid=(NR + 1, NC),
        in_specs=[
            pl.BlockSpec(
                (RB, CB),
                lambda r, c: (jnp.minimum(r, NR - 1),
                              jnp.where(r < NR, c, NC - 1)),
            ),
        ],
        out_specs=[
            pl.BlockSpec(
                (RB, CB),
                lambda r, c: (jnp.maximum(r - 1, 0),
                              jnp.where(r >= 1, c, 0)),
            ),
            pl.BlockSpec(
                (RB, CB),
                lambda r, c: (jnp.maximum(r - 1, 0),
                              jnp.where(r >= 1, c, 0)),
            ),
            pl.BlockSpec((RB, 128),
                         lambda r, c: (jnp.minimum(r, NR - 1), 0)),
            pl.BlockSpec((RB, 128),
                         lambda r, c: (jnp.minimum(r, NR - 1), 0)),
        ],
        out_shape=[
            jax.ShapeDtypeStruct((B, N), jnp.float32),
            jax.ShapeDtypeStruct((B, N), jnp.float32),
            jax.ShapeDtypeStruct((B, 128), jnp.int32),
            jax.ShapeDtypeStruct((B, 128), jnp.float32),
        ],
        scratch_shapes=[
            pltpu.VMEM((RB, 1), jnp.float32),
            pltpu.VMEM((RB, 1), jnp.float32),
            pltpu.VMEM((RB, 1), jnp.int32),
        ],
        compiler_params=pltpu.CompilerParams(
            dimension_semantics=("arbitrary", "arbitrary"),
        ),
    )(virtual_values)


def _sc_reduce_body(x_ref, fi_ref, fp_ref, buf0, buf1, sem0, sem1):
    wid = lax.axis_index("s") * SC_CORES + lax.axis_index("c")
    base = wid * RPW
    lane = lax.iota(jnp.int32, SC_LANES)
    ninf = jnp.full((SC_LANES,), NEG_INF, jnp.float32)

    CHUNKS = N // SC_LANES
    UNROLL = 5

    def process(buf, row_local):
        def chunk(t, carry):
            m, s, i = carry
            for u in range(UNROLL):
                cbase = (t * UNROLL + u) * SC_LANES
                x = buf[pl.ds(cbase, SC_LANES)]
                gt = x > m
                s = jnp.maximum(s, jnp.minimum(m, x))
                i = jnp.where(gt, cbase + lane, i)
                m = jnp.maximum(m, x)
            return m, s, i

        m, s, i = lax.fori_loop(0, CHUNKS // UNROLL, chunk,
                                (ninf, ninf, jnp.zeros((SC_LANES,),
                                                       jnp.int32)))
        sk, _ = plsc.sort_key_val(m, s, descending=True)
        keys2 = jnp.where(m == sk[0], i, BIG_I32)
        ik, iv = plsc.sort_key_val(keys2, s)
        imin = ik[0]
        payment = jnp.maximum(jnp.maximum(sk[1], iv[0]), 0.0)
        grp = (row_local // SC_LANES) * SC_LANES
        pos = row_local - grp
        iv = fi_ref[pl.ds(grp, SC_LANES)]
        fi_ref[pl.ds(grp, SC_LANES)] = jnp.where(lane == pos, imin, iv)
        pv = fp_ref[pl.ds(grp, SC_LANES)]
        fp_ref[pl.ds(grp, SC_LANES)] = jnp.where(lane == pos, payment, pv)

    pltpu.make_async_copy(x_ref.at[base], buf0, sem0).start()

    def pair(t, carry):
        pltpu.make_async_copy(x_ref.at[base + 2 * t + 1], buf1, sem1).start()
        pltpu.make_async_copy(x_ref.at[base], buf0, sem0).wait()
        process(buf0, 2 * t)

        @pl.when(t < RPW // 2 - 1)
        def _prefetch():
            pltpu.make_async_copy(x_ref.at[base + 2 * t + 2], buf0,
                                  sem0).start()

        pltpu.make_async_copy(x_ref.at[base], buf1, sem1).wait()
        process(buf1, 2 * t + 1)
        return carry

    lax.fori_loop(0, RPW // 2, pair, 0)


def _sc_reduce(virtual_values):
    mesh = plsc.VectorSubcoreMesh(core_axis_name="c", subcore_axis_name="s")

    def body(x_ref, fi_ref, fp_ref, buf0, buf1, fiv, fpv, sem0, sem1):
        wid = lax.axis_index("s") * SC_CORES + lax.axis_index("c")
        _sc_reduce_body(x_ref, fiv, fpv, buf0, buf1, sem0, sem1)
        pltpu.sync_copy(fiv, fi_ref.at[wid])
        pltpu.sync_copy(fpv, fp_ref.at[wid])

    return pl.kernel(
        body,
        out_type=[
            jax.ShapeDtypeStruct((NW, RPW), jnp.int32),
            jax.ShapeDtypeStruct((NW, RPW), jnp.float32),
        ],
        mesh=mesh,
        scratch_types=[
            pltpu.VMEM((N,), jnp.float32),
            pltpu.VMEM((N,), jnp.float32),
            pltpu.VMEM((RPW,), jnp.int32),
            pltpu.VMEM((RPW,), jnp.float32),
            pltpu.SemaphoreType.DMA,
            pltpu.SemaphoreType.DMA,
        ],
        compiler_params=pltpu.CompilerParams(needs_layout_passes=False),
    )(virtual_values)


def _tc_zero_fill():
    def _zero_body(a_ref, p_ref):
        a_ref[...] = jnp.zeros_like(a_ref)
        p_ref[...] = jnp.zeros_like(p_ref)

    return pl.pallas_call(
        _zero_body,
        grid=(NR, NC),
        out_specs=[
            pl.BlockSpec((RB, CB), lambda r, c: (r, c)),
            pl.BlockSpec((RB, CB), lambda r, c: (r, c)),
        ],
        out_shape=[
            jax.ShapeDtypeStruct((B, N), jnp.float32),
            jax.ShapeDtypeStruct((B, N), jnp.float32),
        ],
    )()


def _sc_scatter(alloc_flat, pay_flat, fi, fp):
    mesh = plsc.VectorSubcoreMesh(core_axis_name="c", subcore_axis_name="s")

    def stateful(refs):
        alloc_ref, pay_ref, fi_ref, fp_ref = refs

        @pl.core_map(mesh)
        def _():
            wid = lax.axis_index("s") * SC_CORES + lax.axis_index("c")
            base = wid * RPW

            def inner(fib, fpb, stage_a, stage_p, sem):
                pltpu.async_copy(fi_ref.at[wid], fib, sem).wait()
                pltpu.async_copy(fp_ref.at[wid], fpb, sem).wait()
                lane = lax.iota(jnp.int32, SC_LANES)
                handles = []
                for k in range(RPW // SC_LANES):
                    wv = fib[pl.ds(k * SC_LANES, SC_LANES)]
                    pv = fpb[pl.ds(k * SC_LANES, SC_LANES)]
                    for j in range(SC_LANES):
                        jl = k * SC_LANES + j
                        col = wv[j]
                        w16 = (col // SC_LANES) * SC_LANES
                        off = col - w16
                        sl = pl.ds(jl * SC_LANES, SC_LANES)
                        stage_a[sl] = jnp.where(lane == off, 1.0,
                                                0.0).astype(jnp.float32)
                        stage_p[sl] = jnp.where(lane == off, pv[j],
                                                0.0).astype(jnp.float32)
                        row = base + jl
                        dst = pl.ds(w16, SC_LANES)
                        handles.append(pltpu.async_copy(
                            stage_a.at[sl], alloc_ref.at[row, dst], sem))
                        handles.append(pltpu.async_copy(
                            stage_p.at[sl], pay_ref.at[row, dst], sem))
                for h in handles:
                    h.wait()

            pl.run_scoped(
                inner,
                pltpu.VMEM((RPW,), jnp.int32),
                pltpu.VMEM((RPW,), jnp.float32),
                pltpu.VMEM((RPW * SC_LANES,), jnp.float32),
                pltpu.VMEM((RPW * SC_LANES,), jnp.float32),
                pltpu.SemaphoreType.DMA,
            )

    return pl.run_state(stateful)((alloc_flat, pay_flat, fi, fp))


@jax.jit
def kernel(virtual_values):
    fi_t, fp_t = _sc_reduce(virtual_values)
    alloc_z, pay_z = _tc_zero_fill()
    allocations, payments, _, _ = _sc_scatter(alloc_z, pay_z, fi_t, fp_t)
    return (allocations, payments)
